# baseline (speedup 1.0000x reference)
import functools

import jax
import jax.numpy as jnp
from jax import lax
from jax.experimental import pallas as pl
from jax.experimental.pallas import tpu as pltpu
from jax.experimental.pallas import tpu_sc as plsc

D = 512
NN = 4096
CH = 1024
NC = NN // CH

_SC_INFO = plsc.get_sparse_core_info()
_NW = _SC_INFO.num_cores * _SC_INFO.num_subcores
_PER_W = NN // _NW


def _sc_classify_body(idx_hbm, cell_hbm, cnt_hbm, idx_v, cell_v, cnt_v):
    wid = lax.axis_index("s") * _SC_INFO.num_cores + lax.axis_index("c")
    base = wid * _PER_W
    pltpu.sync_copy(idx_hbm.at[pl.ds(base, _PER_W)], idx_v)
    pltpu.sync_copy(cell_hbm, cell_v)
    c729 = jnp.full((16,), 729, jnp.int32)
    c27 = jnp.full((16,), 27, jnp.int32)
    c4 = jnp.full((16,), 4, jnp.int32)
    c36 = jnp.full((16,), 36, jnp.int32)
    one = jnp.ones((16,), jnp.int32)
    zero = jnp.zeros((16,), jnp.int32)
    cc = cell_v[...]
    ccx = lax.div(cc, c729)
    r2 = cc - c729 * ccx
    ccy = lax.div(r2, c27)
    ccz = r2 - c27 * ccy
    lcv = jnp.zeros((16,), jnp.int32)
    fcv = jnp.zeros((16,), jnp.int32)
    dcv = jnp.zeros((16,), jnp.int32)
    for j in range(_PER_W // 16):
        iv = idx_v[pl.ds(j * 16, 16)]
        ix = lax.div(iv, c729)
        r1 = iv - c729 * ix
        iy = lax.div(r1, c27)
        iz = r1 - c27 * iy
        dx = ix - ccx
        dy = iy - ccy
        dz = iz - ccz
        d2 = dx * dx + dy * dy + dz * dz
        lmask = jnp.minimum(jnp.maximum(c4 - d2, zero), one)
        dmask = jnp.minimum(jnp.maximum(d2 - c36, zero), one)
        lcv = lcv + lmask
        dcv = dcv + dmask
        fcv = fcv + (one - lmask - dmask)
    cnt_v[pl.ds(0, 16)] = lcv.astype(jnp.float32)
    cnt_v[pl.ds(16, 16)] = fcv.astype(jnp.float32)
    cnt_v[pl.ds(32, 16)] = dcv.astype(jnp.float32)
    pltpu.sync_copy(cnt_v, cnt_hbm.at[wid])


def _sc_classify(neighbor_indices_i32, cell_rep):
    mesh = plsc.VectorSubcoreMesh(core_axis_name="c", subcore_axis_name="s")
    k = functools.partial(
        pl.kernel,
        mesh=mesh,
        out_type=jax.ShapeDtypeStruct((_NW, 48), jnp.float32),
        scratch_types=[
            pltpu.VMEM((_PER_W,), jnp.int32),
            pltpu.VMEM((16,), jnp.int32),
            pltpu.VMEM((48,), jnp.float32),
        ],
    )(_sc_classify_body)
    return k(neighbor_indices_i32, cell_rep)


def _body(cell_ref, idx_ref, cs_ref, bm_ref, bl_ref, bu_ref, b1_ref, b2_ref,
          Wg_ref, bg_ref, cnt_ref, ns_hbm, Wm_hbm, Wl_hbm, Wu_hbm, W1_hbm, W2_hbm,
          out_ref,
          ns_v, Wm_v, Wmb_v, Wl_v, Wu_v, W1_v, W2_v,
          ns_sem, wm_sem, wl_sem, wu_sem, w1_sem, w2_sem):
    ns_cp = [pltpu.make_async_copy(ns_hbm.at[pl.ds(c * CH, CH), :],
                                   ns_v.at[pl.ds(c * CH, CH), :],
                                   ns_sem.at[c]) for c in range(NC)]
    wm_cp = pltpu.make_async_copy(Wm_hbm, Wm_v, wm_sem)
    w_cp = [pltpu.make_async_copy(h, v, s) for h, v, s in
            ((Wl_hbm, Wl_v, wl_sem), (Wu_hbm, Wu_v, wu_sem),
             (W1_hbm, W1_v, w1_sem), (W2_hbm, W2_v, w2_sem))]

    ns_cp[0].start()
    wm_cp.start()
    ns_cp[1].start()

    cell = cell_ref[0]
    cx = (cell // 729).astype(jnp.float32)
    cy = ((cell // 27) % 27).astype(jnp.float32)
    cz = (cell % 27).astype(jnp.float32)

    wm_cp.wait()
    Wmb_v[...] = Wm_v[...].astype(jnp.bfloat16)

    local_sum = jnp.zeros((1, D), jnp.float32)
    dist_sum = jnp.zeros((1, D), jnp.float32)
    all_sum = jnp.zeros((1, D), jnp.float32)
    func_sum = jnp.zeros((1, D), jnp.float32)
    cnt = cnt_ref[...]
    lc = jnp.sum(cnt[:, 0:16])
    fc = jnp.sum(cnt[:, 16:32])
    dc = jnp.sum(cnt[:, 32:48])

    for c in range(NC):
        idx = idx_ref[c].astype(jnp.float32)
        nx = jnp.floor(idx * (1.0 / 729.0))
        r = idx - 729.0 * nx
        ny = jnp.floor(r * (1.0 / 27.0))
        nz = r - 27.0 * ny
        d2 = (nx - cx) ** 2 + (ny - cy) ** 2 + (nz - cz) ** 2
        local_m = jnp.where(d2 <= 3.24, 1.0, 0.0)
        dist_m = jnp.where(d2 > 36.0, 1.0, 0.0)
        func_m = 1.0 - local_m - dist_m

        row = jax.lax.broadcasted_iota(jnp.int32, (8, CH), 0)
        M = jnp.where(row == 0, local_m, jnp.where(row == 1, func_m,
            jnp.where(row == 2, dist_m, 0.0)))
        Mt = jnp.transpose(M, (1, 0))
        lm_col = Mt[:, 0:1]
        fm_col = Mt[:, 1:2]
        dm_col = Mt[:, 2:3]

        ns_cp[c].wait()
        if c + 2 < NC:
            ns_cp[c + 2].start()
        if c == 0:
            w_cp[0].start()
            w_cp[1].start()
        elif c == 1:
            w_cp[2].start()
            w_cp[3].start()
        ns = ns_v[pl.ds(c * CH, CH), :]

        local_sum += jnp.sum(ns * lm_col, axis=0, keepdims=True)
        dist_sum += jnp.sum(ns * dm_col, axis=0, keepdims=True)
        all_sum += jnp.sum(ns, axis=0, keepdims=True)

        t = jnp.tanh(jax.lax.dot_general(
            ns.astype(jnp.bfloat16), Wmb_v[...], (((1,), (0,)), ((), ())),
            preferred_element_type=jnp.float32) + bm_ref[...])
        func_sum += jnp.sum(t * fm_col, axis=0, keepdims=True)

    local_agg = local_sum / jnp.maximum(lc, 1.0)
    func_agg = func_sum / jnp.maximum(fc, 1.0)
    dist_agg = dist_sum / jnp.maximum(dc, 1.0)
    all_agg = all_sum * (1.0 / NN)

    cs = cs_ref[...]

    def mm(a, w):
        return jax.lax.dot_general(a, w, (((1,), (0,)), ((), ())),
                                   preferred_element_type=jnp.float32)

    w_cp[0].wait()
    xl = jnp.concatenate([cs, local_agg], axis=1)
    local_out = jnp.tanh(mm(xl, Wl_v[...]) + bl_ref[...])

    w_cp[1].wait()
    xf = jnp.concatenate([cs, func_agg], axis=1)
    func_out = jnp.tanh(mm(xf, Wu_v[...]) + bu_ref[...])

    w_cp[2].wait()
    w_cp[3].wait()
    z = cs
    for _ in range(3):
        h = jnp.tanh(mm(jnp.concatenate([z, dist_agg], axis=1), W1_v[...])
                     + b1_ref[...])
        z = z + 0.3 * (mm(h, W2_v[...]) + b2_ref[...])

    logits = mm(jnp.concatenate([cs, all_agg], axis=1), Wg_ref[...]) + bg_ref[...]
    m = jnp.max(logits, axis=1, keepdims=True)
    e = jnp.exp(logits - m)
    g = e / jnp.sum(e, axis=1, keepdims=True)

    out_ref[...] = (g[:, 0:1] * local_out + g[:, 1:2] * func_out
                    + g[:, 2:3] * z)


def kernel(current_state, neighbor_states, cell_idx, neighbor_indices,
           W_local, b_local, W_msg, b_msg, W_upd, b_upd,
           W_cnf1, b_cnf1, W_cnf2, b_cnf2, W_gate, b_gate):
    cell_i32 = jnp.asarray(cell_idx, dtype=jnp.int32)
    cell = jnp.reshape(cell_i32, (1,))
    idx_i32 = neighbor_indices.astype(jnp.int32)
    idx3 = jnp.reshape(idx_i32, (NC, 1, CH))
    cs = jnp.reshape(current_state, (1, D))
    cnts = _sc_classify(idx_i32, jnp.full((16,), cell_i32, jnp.int32))

    full = lambda shape: pl.BlockSpec(shape, lambda: (0,) * len(shape))
    any_spec = pl.BlockSpec(memory_space=pl.ANY)
    out = pl.pallas_call(
        _body,
        in_specs=[
            pl.BlockSpec(memory_space=pltpu.SMEM),
            full((NC, 1, CH)),
            full((1, D)),
            full((1, D)),
            full((1, D)),
            full((1, D)),
            full((1, 2 * D)),
            full((1, D)),
            full((2 * D, 3)),
            full((1, 3)),
            full((_NW, 48)),
            any_spec,
            any_spec,
            any_spec,
            any_spec,
            any_spec,
            any_spec,
        ],
        out_specs=pl.BlockSpec((1, D), lambda: (0, 0)),
        out_shape=jax.ShapeDtypeStruct((1, D), jnp.float32),
        scratch_shapes=[
            pltpu.VMEM((NN, D), jnp.float32),
            pltpu.VMEM((D, D), jnp.float32),
            pltpu.VMEM((D, D), jnp.bfloat16),
            pltpu.VMEM((2 * D, D), jnp.float32),
            pltpu.VMEM((2 * D, D), jnp.float32),
            pltpu.VMEM((2 * D, 2 * D), jnp.float32),
            pltpu.VMEM((2 * D, D), jnp.float32),
            pltpu.SemaphoreType.DMA((NC,)),
            pltpu.SemaphoreType.DMA,
            pltpu.SemaphoreType.DMA,
            pltpu.SemaphoreType.DMA,
            pltpu.SemaphoreType.DMA,
            pltpu.SemaphoreType.DMA,
        ],
    )(cell, idx3, cs, jnp.reshape(b_msg, (1, D)), jnp.reshape(b_local, (1, D)),
      jnp.reshape(b_upd, (1, D)), jnp.reshape(b_cnf1, (1, 2 * D)),
      jnp.reshape(b_cnf2, (1, D)), W_gate, jnp.reshape(b_gate, (1, 3)),
      cnts, neighbor_states, W_msg, W_local, W_upd, W_cnf1, W_cnf2)
    return jnp.reshape(out, (D,))


# speedup vs baseline: 2.6675x; 2.6675x over previous
import jax
import jax.numpy as jnp
from jax.experimental import pallas as pl
from jax.experimental.pallas import tpu as pltpu

D = 512
NN = 4096
CH = 1024
NC = NN // CH


def _body(cell_ref, idx_ref, cs_ref, bm_ref, bl_ref, bu_ref, b1_ref, b2_ref,
          Wg_ref, bg_ref, ns_hbm, Wm_hbm, Wl_hbm, Wu_hbm, W1_hbm, W2_hbm,
          out_ref,
          ns_v, Wm_v, Wmb_v, Wl_v, Wu_v, W1_v, W2_v,
          ns_sem, wm_sem, wl_sem, wu_sem, w1_sem, w2_sem):
    ns_cp = [pltpu.make_async_copy(ns_hbm.at[pl.ds(c * CH, CH), :],
                                   ns_v.at[pl.ds(c * CH, CH), :],
                                   ns_sem.at[c]) for c in range(NC)]
    wm_cp = pltpu.make_async_copy(Wm_hbm, Wm_v, wm_sem)
    w_cp = [pltpu.make_async_copy(h, v, s) for h, v, s in
            ((Wl_hbm, Wl_v, wl_sem), (Wu_hbm, Wu_v, wu_sem),
             (W1_hbm, W1_v, w1_sem), (W2_hbm, W2_v, w2_sem))]

    ns_cp[0].start()
    wm_cp.start()
    ns_cp[1].start()
    ns_cp[2].start()

    cell = cell_ref[0]
    cx = (cell // 729).astype(jnp.float32)
    cy = ((cell // 27) % 27).astype(jnp.float32)
    cz = (cell % 27).astype(jnp.float32)

    wm_cp.wait()
    Wmb_v[...] = Wm_v[...].astype(jnp.bfloat16)

    local_sum = jnp.zeros((1, D), jnp.float32)
    dist_sum = jnp.zeros((1, D), jnp.float32)
    all_sum = jnp.zeros((1, D), jnp.float32)
    func_sum = jnp.zeros((1, D), jnp.float32)
    lc = 0.0
    fc = 0.0
    dc = 0.0

    for c in range(NC):
        idx = idx_ref[c].astype(jnp.float32)
        nx = jnp.floor(idx * (1.0 / 729.0))
        r = idx - 729.0 * nx
        ny = jnp.floor(r * (1.0 / 27.0))
        nz = r - 27.0 * ny
        d2 = (nx - cx) ** 2 + (ny - cy) ** 2 + (nz - cz) ** 2
        local_m = jnp.where(d2 <= 3.24, 1.0, 0.0)
        dist_m = jnp.where(d2 > 36.0, 1.0, 0.0)
        func_m = 1.0 - local_m - dist_m

        lc += jnp.sum(local_m)
        fc += jnp.sum(func_m)
        dc += jnp.sum(dist_m)

        row = jax.lax.broadcasted_iota(jnp.int32, (8, CH), 0)
        M = jnp.where(row == 0, local_m, jnp.where(row == 1, func_m,
            jnp.where(row == 2, dist_m, 0.0)))
        Mt = jnp.transpose(M, (1, 0))
        lm_col = Mt[:, 0:1]
        fm_col = Mt[:, 1:2]
        dm_col = Mt[:, 2:3]

        ns_cp[c].wait()
        if c + 3 < NC:
            ns_cp[c + 3].start()
        if c == 0:
            w_cp[0].start()
            w_cp[1].start()
        elif c == 1:
            w_cp[2].start()
            w_cp[3].start()
        ns = ns_v[pl.ds(c * CH, CH), :]

        local_sum += jnp.sum(ns * lm_col, axis=0, keepdims=True)
        dist_sum += jnp.sum(ns * dm_col, axis=0, keepdims=True)
        all_sum += jnp.sum(ns, axis=0, keepdims=True)

        t = jnp.tanh(jax.lax.dot_general(
            ns.astype(jnp.bfloat16), Wmb_v[...], (((1,), (0,)), ((), ())),
            preferred_element_type=jnp.float32) + bm_ref[...])
        func_sum += jnp.sum(t * fm_col, axis=0, keepdims=True)

    local_agg = local_sum / jnp.maximum(lc, 1.0)
    func_agg = func_sum / jnp.maximum(fc, 1.0)
    dist_agg = dist_sum / jnp.maximum(dc, 1.0)
    all_agg = all_sum * (1.0 / NN)

    cs = cs_ref[...]

    def mm(a, w):
        return jax.lax.dot_general(a, w, (((1,), (0,)), ((), ())),
                                   preferred_element_type=jnp.float32)

    w_cp[0].wait()
    xl = jnp.concatenate([cs, local_agg], axis=1)
    local_out = jnp.tanh(mm(xl, Wl_v[...]) + bl_ref[...])

    w_cp[1].wait()
    xf = jnp.concatenate([cs, func_agg], axis=1)
    func_out = jnp.tanh(mm(xf, Wu_v[...]) + bu_ref[...])

    w_cp[2].wait()
    w_cp[3].wait()
    z = cs
    for _ in range(3):
        h = jnp.tanh(mm(jnp.concatenate([z, dist_agg], axis=1), W1_v[...])
                     + b1_ref[...])
        z = z + 0.3 * (mm(h, W2_v[...]) + b2_ref[...])

    logits = mm(jnp.concatenate([cs, all_agg], axis=1), Wg_ref[...]) + bg_ref[...]
    m = jnp.max(logits, axis=1, keepdims=True)
    e = jnp.exp(logits - m)
    g = e / jnp.sum(e, axis=1, keepdims=True)

    out_ref[...] = (g[:, 0:1] * local_out + g[:, 1:2] * func_out
                    + g[:, 2:3] * z)


def kernel(current_state, neighbor_states, cell_idx, neighbor_indices,
           W_local, b_local, W_msg, b_msg, W_upd, b_upd,
           W_cnf1, b_cnf1, W_cnf2, b_cnf2, W_gate, b_gate):
    cell = jnp.reshape(jnp.asarray(cell_idx, dtype=jnp.int32), (1,))
    idx3 = jnp.reshape(neighbor_indices.astype(jnp.int32), (NC, 1, CH))
    cs = jnp.reshape(current_state, (1, D))

    full = lambda shape: pl.BlockSpec(shape, lambda: (0,) * len(shape))
    any_spec = pl.BlockSpec(memory_space=pl.ANY)
    out = pl.pallas_call(
        _body,
        in_specs=[
            pl.BlockSpec(memory_space=pltpu.SMEM),
            full((NC, 1, CH)),
            full((1, D)),
            full((1, D)),
            full((1, D)),
            full((1, D)),
            full((1, 2 * D)),
            full((1, D)),
            full((2 * D, 3)),
            full((1, 3)),
            any_spec,
            any_spec,
            any_spec,
            any_spec,
            any_spec,
            any_spec,
        ],
        out_specs=pl.BlockSpec((1, D), lambda: (0, 0)),
        out_shape=jax.ShapeDtypeStruct((1, D), jnp.float32),
        scratch_shapes=[
            pltpu.VMEM((NN, D), jnp.float32),
            pltpu.VMEM((D, D), jnp.float32),
            pltpu.VMEM((D, D), jnp.bfloat16),
            pltpu.VMEM((2 * D, D), jnp.float32),
            pltpu.VMEM((2 * D, D), jnp.float32),
            pltpu.VMEM((2 * D, 2 * D), jnp.float32),
            pltpu.VMEM((2 * D, D), jnp.float32),
            pltpu.SemaphoreType.DMA((NC,)),
            pltpu.SemaphoreType.DMA,
            pltpu.SemaphoreType.DMA,
            pltpu.SemaphoreType.DMA,
            pltpu.SemaphoreType.DMA,
            pltpu.SemaphoreType.DMA,
        ],
    )(cell, idx3, cs, jnp.reshape(b_msg, (1, D)), jnp.reshape(b_local, (1, D)),
      jnp.reshape(b_upd, (1, D)), jnp.reshape(b_cnf1, (1, 2 * D)),
      jnp.reshape(b_cnf2, (1, D)), W_gate, jnp.reshape(b_gate, (1, 3)),
      neighbor_states, W_msg, W_local, W_upd, W_cnf1, W_cnf2)
    return jnp.reshape(out, (D,))


# speedup vs baseline: 2.7293x; 1.0232x over previous
import jax
import jax.numpy as jnp
from jax.experimental import pallas as pl
from jax.experimental.pallas import tpu as pltpu

D = 512
NN = 4096
CH = 1024
NC = NN // CH


def _body(cell_ref, idx_ref, cs_ref, bm_ref, bl_ref, bu_ref, b1_ref, b2_ref,
          Wg_ref, bg_ref, ns_hbm, Wm_hbm, Wl_hbm, Wu_hbm, W1_hbm, W2_hbm,
          out_ref,
          ns_v, Wm_v, Wmb_v, Wl_v, Wu_v, W1_v, W2_v, mt_v,
          ns_sem, wm_sem, wl_sem, wu_sem, w1_sem, w2_sem):
    ns_cp = [pltpu.make_async_copy(ns_hbm.at[pl.ds(c * CH, CH), :],
                                   ns_v.at[pl.ds(c * CH, CH), :],
                                   ns_sem.at[c]) for c in range(NC)]
    wm_cp = pltpu.make_async_copy(Wm_hbm, Wm_v, wm_sem)
    w_cp = [pltpu.make_async_copy(h, v, s) for h, v, s in
            ((Wl_hbm, Wl_v, wl_sem), (Wu_hbm, Wu_v, wu_sem),
             (W1_hbm, W1_v, w1_sem), (W2_hbm, W2_v, w2_sem))]

    ns_cp[0].start()
    wm_cp.start()
    ns_cp[1].start()
    ns_cp[2].start()

    cell = cell_ref[0]
    cx = (cell // 729).astype(jnp.float32)
    cy = ((cell // 27) % 27).astype(jnp.float32)
    cz = (cell % 27).astype(jnp.float32)

    local_sum = jnp.zeros((1, D), jnp.float32)
    dist_sum = jnp.zeros((1, D), jnp.float32)
    all_sum = jnp.zeros((1, D), jnp.float32)
    func_sum = jnp.zeros((1, D), jnp.float32)
    lc = 0.0
    fc = 0.0
    dc = 0.0

    row = jax.lax.broadcasted_iota(jnp.int32, (8, CH), 0)
    for c in range(NC):
        idx = idx_ref[c].astype(jnp.float32)
        nx = jnp.floor(idx * (1.0 / 729.0))
        r = idx - 729.0 * nx
        ny = jnp.floor(r * (1.0 / 27.0))
        nz = r - 27.0 * ny
        d2 = (nx - cx) ** 2 + (ny - cy) ** 2 + (nz - cz) ** 2
        local_m = jnp.where(d2 <= 3.24, 1.0, 0.0)
        dist_m = jnp.where(d2 > 36.0, 1.0, 0.0)
        func_m = 1.0 - local_m - dist_m

        lc += jnp.sum(local_m)
        fc += jnp.sum(func_m)
        dc += jnp.sum(dist_m)

        M = jnp.where(row == 0, local_m, jnp.where(row == 1, func_m,
            jnp.where(row == 2, dist_m, 0.0)))
        mt_v[pl.ds(c * CH, CH), :] = jnp.transpose(M, (1, 0))

    wm_cp.wait()
    Wmb_v[...] = Wm_v[...].astype(jnp.bfloat16)

    for c in range(NC):
        Mt = mt_v[pl.ds(c * CH, CH), :]
        lm_col = Mt[:, 0:1]
        fm_col = Mt[:, 1:2]
        dm_col = Mt[:, 2:3]

        ns_cp[c].wait()
        if c + 3 < NC:
            ns_cp[c + 3].start()
        if c == 0:
            w_cp[0].start()
            w_cp[1].start()
        elif c == 1:
            w_cp[2].start()
            w_cp[3].start()
        ns = ns_v[pl.ds(c * CH, CH), :]

        local_sum += jnp.sum(ns * lm_col, axis=0, keepdims=True)
        dist_sum += jnp.sum(ns * dm_col, axis=0, keepdims=True)
        all_sum += jnp.sum(ns, axis=0, keepdims=True)

        t = jnp.tanh(jax.lax.dot_general(
            ns.astype(jnp.bfloat16), Wmb_v[...], (((1,), (0,)), ((), ())),
            preferred_element_type=jnp.float32) + bm_ref[...])
        func_sum += jnp.sum(t * fm_col, axis=0, keepdims=True)

    local_agg = local_sum / jnp.maximum(lc, 1.0)
    func_agg = func_sum / jnp.maximum(fc, 1.0)
    dist_agg = dist_sum / jnp.maximum(dc, 1.0)
    all_agg = all_sum * (1.0 / NN)

    cs = cs_ref[...]

    def mm(a, w):
        return jax.lax.dot_general(a, w, (((1,), (0,)), ((), ())),
                                   preferred_element_type=jnp.float32)

    w_cp[0].wait()
    xl = jnp.concatenate([cs, local_agg], axis=1)
    local_out = jnp.tanh(mm(xl, Wl_v[...]) + bl_ref[...])

    w_cp[1].wait()
    xf = jnp.concatenate([cs, func_agg], axis=1)
    func_out = jnp.tanh(mm(xf, Wu_v[...]) + bu_ref[...])

    w_cp[2].wait()
    w_cp[3].wait()
    z = cs
    for _ in range(3):
        h = jnp.tanh(mm(jnp.concatenate([z, dist_agg], axis=1), W1_v[...])
                     + b1_ref[...])
        z = z + 0.3 * (mm(h, W2_v[...]) + b2_ref[...])

    logits = mm(jnp.concatenate([cs, all_agg], axis=1), Wg_ref[...]) + bg_ref[...]
    m = jnp.max(logits, axis=1, keepdims=True)
    e = jnp.exp(logits - m)
    g = e / jnp.sum(e, axis=1, keepdims=True)

    out_ref[...] = (g[:, 0:1] * local_out + g[:, 1:2] * func_out
                    + g[:, 2:3] * z)


def kernel(current_state, neighbor_states, cell_idx, neighbor_indices,
           W_local, b_local, W_msg, b_msg, W_upd, b_upd,
           W_cnf1, b_cnf1, W_cnf2, b_cnf2, W_gate, b_gate):
    cell = jnp.reshape(jnp.asarray(cell_idx, dtype=jnp.int32), (1,))
    idx3 = jnp.reshape(neighbor_indices.astype(jnp.int32), (NC, 1, CH))
    cs = jnp.reshape(current_state, (1, D))

    full = lambda shape: pl.BlockSpec(shape, lambda: (0,) * len(shape))
    any_spec = pl.BlockSpec(memory_space=pl.ANY)
    out = pl.pallas_call(
        _body,
        in_specs=[
            pl.BlockSpec(memory_space=pltpu.SMEM),
            full((NC, 1, CH)),
            full((1, D)),
            full((1, D)),
            full((1, D)),
            full((1, D)),
            full((1, 2 * D)),
            full((1, D)),
            full((2 * D, 3)),
            full((1, 3)),
            any_spec,
            any_spec,
            any_spec,
            any_spec,
            any_spec,
            any_spec,
        ],
        out_specs=pl.BlockSpec((1, D), lambda: (0, 0)),
        out_shape=jax.ShapeDtypeStruct((1, D), jnp.float32),
        scratch_shapes=[
            pltpu.VMEM((NN, D), jnp.float32),
            pltpu.VMEM((D, D), jnp.float32),
            pltpu.VMEM((D, D), jnp.bfloat16),
            pltpu.VMEM((2 * D, D), jnp.float32),
            pltpu.VMEM((2 * D, D), jnp.float32),
            pltpu.VMEM((2 * D, 2 * D), jnp.float32),
            pltpu.VMEM((2 * D, D), jnp.float32),
            pltpu.VMEM((NN, 8), jnp.float32),
            pltpu.SemaphoreType.DMA((NC,)),
            pltpu.SemaphoreType.DMA,
            pltpu.SemaphoreType.DMA,
            pltpu.SemaphoreType.DMA,
            pltpu.SemaphoreType.DMA,
            pltpu.SemaphoreType.DMA,
        ],
    )(cell, idx3, cs, jnp.reshape(b_msg, (1, D)), jnp.reshape(b_local, (1, D)),
      jnp.reshape(b_upd, (1, D)), jnp.reshape(b_cnf1, (1, 2 * D)),
      jnp.reshape(b_cnf2, (1, D)), W_gate, jnp.reshape(b_gate, (1, 3)),
      neighbor_states, W_msg, W_local, W_upd, W_cnf1, W_cnf2)
    return jnp.reshape(out, (D,))
